# Initial kernel scaffold; baseline (speedup 1.0000x reference)
import jax
import jax.numpy as jnp
from jax.experimental import pallas as pl


def kernel(edge_index, edge_type, init_embed, init_rel, w_rel, enc_Wrel, enc_Wroot, enc_b, lin0_w, lin0_b, lin1_w, lin1_b, dec_init_embed, dec_init_rel, dec_w_rel, dec_Wrel, dec_Wroot, dec_b):
    raise NotImplementedError("write your pallas kernel here")


# speedup vs baseline: 1.9756x; 1.9756x over previous
import jax
import jax.numpy as jnp
from jax import lax
from jax.experimental import pallas as pl
from jax.experimental.pallas import tpu as pltpu
from jax.experimental.pallas import tpu_sc as plsc

N = 10000
E = 320000
D = 128
R2 = 16
NC = 2
NS = 16
NW = NC * NS
EPW = E // NW
CH = 80
NSC = 5
SCE = EPW // NSC
NCH = SCE // CH
NB = 25
NBR = N // NB
RPT = 624

f32 = jnp.float32
i32 = jnp.int32

_SC_PARAMS = dict(
    compiler_params=pltpu.CompilerParams(needs_layout_passes=False))


def _k1a_body(ie_ref, ew_ref, de_ref, dw_ref, oute_ref, outd_ref):
    oute_ref[...] = jnp.dot(ie_ref[...], ew_ref[0], preferred_element_type=f32)
    outd_ref[...] = jnp.dot(de_ref[...], dw_ref[0], preferred_element_type=f32)


def _k1a(init_embed, enc_Wrel, dec_init_embed, dec_Wrel):
    return pl.pallas_call(
        _k1a_body,
        grid=(R2, NB),
        in_specs=[
            pl.BlockSpec((NBR, D), lambda r, n: (n, 0)),
            pl.BlockSpec((1, D, D), lambda r, n: (r, 0, 0)),
            pl.BlockSpec((NBR, D), lambda r, n: (n, 0)),
            pl.BlockSpec((1, D, D), lambda r, n: (r, 0, 0)),
        ],
        out_specs=[
            pl.BlockSpec((NBR, D), lambda r, n: (r * NB + n, 0)),
            pl.BlockSpec((NBR, D), lambda r, n: (r * NB + n, 0)),
        ],
        out_shape=[
            jax.ShapeDtypeStruct((R2 * N, D), f32),
            jax.ShapeDtypeStruct((R2 * N, D), f32),
        ],
    )(init_embed, enc_Wrel, dec_init_embed, dec_Wrel)


def _k1c_body(u0_ref, u1_ref, ir_ref, wr_ref, dir_ref, dwr_ref,
              gthr_ref, rdec_ref, renc_ref):
    g0 = -jnp.log(-jnp.log(u0_ref[...]))
    g1 = -jnp.log(-jnp.log(u1_ref[...]))
    gthr_ref[...] = g1 - g0
    renc_ref[...] = jnp.dot(ir_ref[...], wr_ref[...],
                            preferred_element_type=f32)
    rdec_ref[...] = jnp.dot(dir_ref[...], dwr_ref[...],
                            preferred_element_type=f32)


def _k1c(u0, u1, init_rel, w_rel, dec_init_rel, dec_w_rel):
    return pl.pallas_call(
        _k1c_body,
        out_shape=[
            jax.ShapeDtypeStruct((E // D, D), f32),
            jax.ShapeDtypeStruct((R2, D), f32),
            jax.ShapeDtypeStruct((R2, D), f32),
        ],
    )(u0, u1, init_rel, w_rel, dec_init_rel, dec_w_rel)


def _k3_body(p0_ref, p1_ref, ie_ref, wroot_ref, b_ref, x_ref):
    x_ref[...] = (p0_ref[...] + p1_ref[...] + b_ref[...]
                  + jnp.dot(ie_ref[...], wroot_ref[...],
                            preferred_element_type=f32))


def _k3(p0, p1, init_embed, enc_Wroot, enc_b):
    full = lambda shp: pl.BlockSpec(shp, lambda n: tuple(0 for _ in shp))
    return pl.pallas_call(
        _k3_body,
        grid=(NB,),
        in_specs=[
            pl.BlockSpec((NBR, D), lambda n: (n, 0)),
            pl.BlockSpec((NBR, D), lambda n: (n, 0)),
            pl.BlockSpec((NBR, D), lambda n: (n, 0)),
            full((D, D)),
            full((1, D)),
        ],
        out_specs=pl.BlockSpec((NBR, D), lambda n: (n, 0)),
        out_shape=jax.ShapeDtypeStruct((N, D), f32),
    )(p0, p1, init_embed, enc_Wroot, enc_b)


def _k5_body(p0_ref, p1_ref, de_ref, wroot_ref, b_ref, out_ref):
    out_ref[...] = (p0_ref[...] + p1_ref[...] + b_ref[...]
                    + jnp.dot(de_ref[...], wroot_ref[...],
                              preferred_element_type=f32))


def _k5(p0, p1, dec_init_embed, dec_Wroot, dec_b):
    full = lambda shp: pl.BlockSpec(shp, lambda n: tuple(0 for _ in shp))
    return pl.pallas_call(
        _k5_body,
        grid=(NB,),
        in_specs=[
            pl.BlockSpec((NBR, D), lambda n: (n, 0)),
            pl.BlockSpec((NBR, D), lambda n: (n, 0)),
            pl.BlockSpec((NBR, D), lambda n: (n, 0)),
            full((D, D)),
            full((1, D)),
        ],
        out_specs=pl.BlockSpec((NBR, D), lambda n: (n, 0)),
        out_shape=jax.ShapeDtypeStruct((N, D), f32),
    )(p0, p1, dec_init_embed, dec_Wroot, dec_b)


def _zero_rows(buf_ref):
    z = jnp.zeros((16,), f32)

    @pl.loop(0, CH)
    def _(r):
        for k in range(D // 16):
            buf_ref[r, pl.ds(k * 16, 16)] = z


def _zero_agg(agg_ref, zbuf_ref, s):
    base = s * RPT

    @pl.loop(0, 7)
    def _(t):
        pltpu.sync_copy(zbuf_ref, agg_ref.at[pl.ds(base + t * CH, CH)])

    pltpu.sync_copy(zbuf_ref.at[pl.ds(0, RPT - 7 * CH)],
                    agg_ref.at[pl.ds(base + 7 * CH, RPT - 7 * CH)])

    @pl.when(s == 0)
    def _():
        pltpu.sync_copy(zbuf_ref.at[pl.ds(0, N - NS * RPT)],
                        agg_ref.at[pl.ds(NS * RPT, N - NS * RPT)])


def _writeout_agg(agg_ref, out_hbm, cc, s):
    base = s * RPT
    pltpu.sync_copy(agg_ref.at[pl.ds(base, RPT)],
                    out_hbm.at[cc, pl.ds(base, RPT)])

    @pl.when(s == 0)
    def _():
        pltpu.sync_copy(agg_ref.at[pl.ds(NS * RPT, N - NS * RPT)],
                        out_hbm.at[cc, pl.ds(NS * RPT, N - NS * RPT)])


def _compute_fidx(t1, s1, f1):
    @pl.loop(0, SCE // 16)
    def _(i):
        f1[pl.ds(i * 16, 16)] = (t1[pl.ds(i * 16, 16)] * N
                                 + s1[pl.ds(i * 16, 16)])


def _conv_body(scale, xrel_hbm, src3f_hbm, typ3f_hbm, dst3_hbm, sc3f_hbm,
               out_hbm, s1, t1, f1, d2, scb, rows, agg, sem):
    cc = lax.axis_index("c")
    s = lax.axis_index("s")
    wid = cc * NS + s

    _zero_rows(rows)
    _zero_agg(agg, rows, s)
    plsc.subcore_barrier()

    @pl.loop(0, NSC)
    def _(u):
        pltpu.sync_copy(src3f_hbm.at[wid, u], s1)
        pltpu.sync_copy(typ3f_hbm.at[wid, u], t1)
        pltpu.sync_copy(dst3_hbm.at[wid, u], d2)
        if scale:
            pltpu.sync_copy(sc3f_hbm.at[wid, u], scb)
        _compute_fidx(t1, s1, f1)

        @pl.loop(0, NCH)
        def _(c):
            pltpu.async_copy(xrel_hbm.at[f1.at[pl.ds(c * CH, CH)]],
                             rows, sem).wait()
            if scale:
                @pl.loop(0, CH)
                def _(e):
                    ssp = plsc.load_gather(scb, [jnp.full((16,), c * CH + e,
                                                          i32)])
                    for j in range(D // 16):
                        rows[e, pl.ds(j * 16, 16)] = (
                            rows[e, pl.ds(j * 16, 16)] * ssp)
            pltpu.sync_copy(rows, agg.at[d2.at[c]], add=True)

    plsc.subcore_barrier()
    _writeout_agg(agg, out_hbm, cc, s)


def _conv_sc(xrel, src3f, typ3f, dst3, sc3f, scale):
    mesh = plsc.VectorSubcoreMesh(core_axis_name="c", subcore_axis_name="s")

    def body(xrel_, src_, typ_, dst_, sc_, out_, s1, t1, f1, d2, scb, rows,
             agg, sem):
        _conv_body(scale, xrel_, src_, typ_, dst_, sc_, out_, s1, t1, f1,
                   d2, scb, rows, agg, sem)

    return pl.kernel(
        body,
        out_type=jax.ShapeDtypeStruct((NC, N, D), f32),
        mesh=mesh,
        scratch_types=[
            pltpu.VMEM((SCE,), i32),
            pltpu.VMEM((SCE,), i32),
            pltpu.VMEM((SCE,), i32),
            pltpu.VMEM((NCH, CH), i32),
            pltpu.VMEM((SCE,), f32),
            pltpu.VMEM((CH, D), f32),
            pltpu.VMEM_SHARED((N, D), f32),
            pltpu.SemaphoreType.DMA,
        ],
        **_SC_PARAMS,
    )(xrel, src3f, typ3f, dst3, sc3f)


def _k4g_body(x_hbm, src3f_hbm, dst3f_hbm, hs_hbm, ht_hbm,
              s1, d1, arows, crows, sem, sem2):
    cc = lax.axis_index("c")
    s = lax.axis_index("s")
    wid = cc * NS + s

    @pl.loop(0, NSC)
    def _(u):
        pltpu.sync_copy(src3f_hbm.at[wid, u], s1)
        pltpu.sync_copy(dst3f_hbm.at[wid, u], d1)
        base = wid * EPW + u * SCE

        @pl.loop(0, NCH)
        def _(c):
            cp1 = pltpu.async_copy(x_hbm.at[s1.at[pl.ds(c * CH, CH)]],
                                   arows, sem)
            cp2 = pltpu.async_copy(x_hbm.at[d1.at[pl.ds(c * CH, CH)]],
                                   crows, sem2)
            cp1.wait()
            cp2.wait()
            pltpu.sync_copy(arows, hs_hbm.at[pl.ds(base + c * CH, CH)])
            pltpu.sync_copy(crows, ht_hbm.at[pl.ds(base + c * CH, CH)])


def _k4g(x, src3f, dst3f):
    mesh = plsc.VectorSubcoreMesh(core_axis_name="c", subcore_axis_name="s")
    return pl.kernel(
        _k4g_body,
        out_type=[
            jax.ShapeDtypeStruct((E, D), f32),
            jax.ShapeDtypeStruct((E, D), f32),
        ],
        mesh=mesh,
        scratch_types=[
            pltpu.VMEM((SCE,), i32),
            pltpu.VMEM((SCE,), i32),
            pltpu.VMEM((CH, D), f32),
            pltpu.VMEM((CH, D), f32),
            pltpu.SemaphoreType.DMA,
            pltpu.SemaphoreType.DMA,
        ],
        **_SC_PARAMS,
    )(x, src3f, dst3f)


EB = 3200
NEB = E // EB
EBR = EB // D


def _k4m_body(hs_ref, ht_ref, typ_ref, r_ref, gthr_ref, w0t_ref, b0_ref,
              w1t_ref, b1_ref, score_ref, gsm_ref):
    typ = jnp.min(typ_ref[...], axis=1, keepdims=True)
    r = r_ref[...]
    hr = jnp.zeros((EB, D), f32)
    for k in range(R2):
        hr = jnp.where(typ == k, r[k][None, :], hr)
    h = jnp.concatenate([hs_ref[...], hr, ht_ref[...]], axis=1)
    nrm = jnp.sqrt(jnp.sum(h * h, axis=1, keepdims=True))
    h = h / jnp.maximum(nrm, 1e-12)
    h2 = jax.nn.relu(jnp.dot(h, w0t_ref[...]) + b0_ref[...])
    z = jnp.dot(h2, w1t_ref[...]) + b1_ref[...]
    sc = jax.nn.sigmoid(z)
    gthr_c = jnp.min(gthr_ref[...], axis=1, keepdims=True)
    score_ref[...] = jnp.broadcast_to(sc, (EB, 8))
    gsm_ref[...] = jnp.broadcast_to(
        jnp.where(2.0 * sc - 1.0 >= gthr_c, 1.0, 0.0), (EB, 8))


def _k4m(hs, ht, typ8, r_enc, gthr8, w0t, b0, w1t, b1):
    full = lambda shp: pl.BlockSpec(shp, lambda n: tuple(0 for _ in shp))
    return pl.pallas_call(
        _k4m_body,
        grid=(NEB,),
        in_specs=[
            pl.BlockSpec((EB, D), lambda n: (n, 0)),
            pl.BlockSpec((EB, D), lambda n: (n, 0)),
            pl.BlockSpec((EB, 8), lambda n: (n, 0)),
            full((R2, D)),
            pl.BlockSpec((EB, 8), lambda n: (n, 0)),
            full((3 * D, D)),
            full((1, D)),
            full((D, 1)),
            full((1, 1)),
        ],
        out_specs=[
            pl.BlockSpec((EB, 8), lambda n: (n, 0)),
            pl.BlockSpec((EB, 8), lambda n: (n, 0)),
        ],
        out_shape=[
            jax.ShapeDtypeStruct((E, 8), f32),
            jax.ShapeDtypeStruct((E, 8), f32),
        ],
    )(hs, ht, typ8, r_enc, gthr8, w0t, b0, w1t, b1)


def kernel(edge_index, edge_type, init_embed, init_rel, w_rel, enc_Wrel,
           enc_Wroot, enc_b, lin0_w, lin0_b, lin1_w, lin1_b, dec_init_embed,
           dec_init_rel, dec_w_rel, dec_Wrel, dec_Wroot, dec_b):
    src3f = edge_index[0].reshape(NW, NSC, SCE)
    dst3f = edge_index[1].reshape(NW, NSC, SCE)
    typ3f = edge_type.reshape(NW, NSC, SCE)
    dst3 = edge_index[1].reshape(NW, NSC, NCH, CH)

    u = jax.random.uniform(jax.random.key(12345), (2, E, 1),
                           minval=1e-6, maxval=1.0 - 1e-6)
    u0 = u[0, :, 0].reshape(E // D, D)
    u1 = u[1, :, 0].reshape(E // D, D)

    xrel_enc, xrel_dec = _k1a(init_embed, enc_Wrel, dec_init_embed, dec_Wrel)
    gthr, r_, r_enc = _k1c(u0, u1, init_rel, w_rel, dec_init_rel, dec_w_rel)

    dummy_sc = jnp.zeros((NW, NSC, SCE), f32)
    parts_e = _conv_sc(xrel_enc, src3f, typ3f, dst3, dummy_sc, scale=False)
    x = _k3(parts_e[0], parts_e[1], init_embed, enc_Wroot,
            enc_b.reshape(1, D))

    hs, ht = _k4g(x, src3f, dst3f)
    typ8 = jnp.broadcast_to(edge_type.reshape(E, 1), (E, 8))
    gthr8 = jnp.broadcast_to(gthr.reshape(E, 1), (E, 8))
    score2, gsm2 = _k4m(hs, ht, typ8, r_enc, gthr8,
                        lin0_w.T, lin0_b.reshape(1, D), lin1_w.T,
                        lin1_b.reshape(1, 1))
    score3f = score2[:, 0].reshape(NW, NSC, SCE)

    parts_d = _conv_sc(xrel_dec, src3f, typ3f, dst3, score3f, scale=True)

    x_ = _k5(parts_d[0], parts_d[1], dec_init_embed, dec_Wroot,
             dec_b.reshape(1, D))

    return (x_, r_, gsm2[:, :1], score2[:, :1])


# speedup vs baseline: 2.3008x; 1.1646x over previous
import jax
import jax.numpy as jnp
from jax import lax
from jax.experimental import pallas as pl
from jax.experimental.pallas import tpu as pltpu
from jax.experimental.pallas import tpu_sc as plsc

N = 10000
E = 320000
D = 128
R2 = 16
NC = 2
NS = 16
NW = NC * NS
EPW = E // NW
CH = 80
NSC = 5
SCE = EPW // NSC
NCH = SCE // CH
NB = 25
NBR = N // NB
RPT = 624

f32 = jnp.float32
i32 = jnp.int32

_SC_PARAMS = dict(
    compiler_params=pltpu.CompilerParams(needs_layout_passes=False))


def _k1a_body(ie_ref, ew_ref, de_ref, dw_ref, oute_ref, outd_ref):
    oute_ref[...] = jnp.dot(ie_ref[...], ew_ref[0], preferred_element_type=f32)
    outd_ref[...] = jnp.dot(de_ref[...], dw_ref[0], preferred_element_type=f32)


def _k1a(init_embed, enc_Wrel, dec_init_embed, dec_Wrel):
    return pl.pallas_call(
        _k1a_body,
        grid=(R2, NB),
        in_specs=[
            pl.BlockSpec((NBR, D), lambda r, n: (n, 0)),
            pl.BlockSpec((1, D, D), lambda r, n: (r, 0, 0)),
            pl.BlockSpec((NBR, D), lambda r, n: (n, 0)),
            pl.BlockSpec((1, D, D), lambda r, n: (r, 0, 0)),
        ],
        out_specs=[
            pl.BlockSpec((NBR, D), lambda r, n: (r * NB + n, 0)),
            pl.BlockSpec((NBR, D), lambda r, n: (r * NB + n, 0)),
        ],
        out_shape=[
            jax.ShapeDtypeStruct((R2 * N, D), f32),
            jax.ShapeDtypeStruct((R2 * N, D), f32),
        ],
    )(init_embed, enc_Wrel, dec_init_embed, dec_Wrel)


def _k1c_body(u0_ref, u1_ref, ir_ref, wr_ref, dir_ref, dwr_ref,
              gthr_ref, rdec_ref, renc_ref):
    g0 = -jnp.log(-jnp.log(u0_ref[...]))
    g1 = -jnp.log(-jnp.log(u1_ref[...]))
    gthr_ref[...] = g1 - g0
    renc_ref[...] = jnp.dot(ir_ref[...], wr_ref[...],
                            preferred_element_type=f32)
    rdec_ref[...] = jnp.dot(dir_ref[...], dwr_ref[...],
                            preferred_element_type=f32)


def _k1c(u0, u1, init_rel, w_rel, dec_init_rel, dec_w_rel):
    return pl.pallas_call(
        _k1c_body,
        out_shape=[
            jax.ShapeDtypeStruct((E // D, D), f32),
            jax.ShapeDtypeStruct((R2, D), f32),
            jax.ShapeDtypeStruct((R2, D), f32),
        ],
    )(u0, u1, init_rel, w_rel, dec_init_rel, dec_w_rel)


def _k3_body(p0_ref, p1_ref, ie_ref, wroot_ref, b_ref, x_ref):
    x_ref[...] = (p0_ref[...] + p1_ref[...] + b_ref[...]
                  + jnp.dot(ie_ref[...], wroot_ref[...],
                            preferred_element_type=f32))


def _k3(p0, p1, init_embed, enc_Wroot, enc_b):
    full = lambda shp: pl.BlockSpec(shp, lambda n: tuple(0 for _ in shp))
    return pl.pallas_call(
        _k3_body,
        grid=(NB,),
        in_specs=[
            pl.BlockSpec((NBR, D), lambda n: (n, 0)),
            pl.BlockSpec((NBR, D), lambda n: (n, 0)),
            pl.BlockSpec((NBR, D), lambda n: (n, 0)),
            full((D, D)),
            full((1, D)),
        ],
        out_specs=pl.BlockSpec((NBR, D), lambda n: (n, 0)),
        out_shape=jax.ShapeDtypeStruct((N, D), f32),
    )(p0, p1, init_embed, enc_Wroot, enc_b)


def _k5_body(p0_ref, p1_ref, de_ref, wroot_ref, b_ref, out_ref):
    out_ref[...] = (p0_ref[...] + p1_ref[...] + b_ref[...]
                    + jnp.dot(de_ref[...], wroot_ref[...],
                              preferred_element_type=f32))


def _k5(p0, p1, dec_init_embed, dec_Wroot, dec_b):
    full = lambda shp: pl.BlockSpec(shp, lambda n: tuple(0 for _ in shp))
    return pl.pallas_call(
        _k5_body,
        grid=(NB,),
        in_specs=[
            pl.BlockSpec((NBR, D), lambda n: (n, 0)),
            pl.BlockSpec((NBR, D), lambda n: (n, 0)),
            pl.BlockSpec((NBR, D), lambda n: (n, 0)),
            full((D, D)),
            full((1, D)),
        ],
        out_specs=pl.BlockSpec((NBR, D), lambda n: (n, 0)),
        out_shape=jax.ShapeDtypeStruct((N, D), f32),
    )(p0, p1, dec_init_embed, dec_Wroot, dec_b)


def _zero_rows(buf_ref):
    z = jnp.zeros((16,), f32)

    @pl.loop(0, CH)
    def _(r):
        for k in range(D // 16):
            buf_ref[r, pl.ds(k * 16, 16)] = z


def _zero_agg(agg_ref, zbuf_ref, s):
    base = s * RPT

    @pl.loop(0, 7)
    def _(t):
        pltpu.sync_copy(zbuf_ref, agg_ref.at[pl.ds(base + t * CH, CH)])

    pltpu.sync_copy(zbuf_ref.at[pl.ds(0, RPT - 7 * CH)],
                    agg_ref.at[pl.ds(base + 7 * CH, RPT - 7 * CH)])

    @pl.when(s == 0)
    def _():
        pltpu.sync_copy(zbuf_ref.at[pl.ds(0, N - NS * RPT)],
                        agg_ref.at[pl.ds(NS * RPT, N - NS * RPT)])


def _writeout_agg(agg_ref, out_hbm, cc, s):
    base = s * RPT
    pltpu.sync_copy(agg_ref.at[pl.ds(base, RPT)],
                    out_hbm.at[cc, pl.ds(base, RPT)])

    @pl.when(s == 0)
    def _():
        pltpu.sync_copy(agg_ref.at[pl.ds(NS * RPT, N - NS * RPT)],
                        out_hbm.at[cc, pl.ds(NS * RPT, N - NS * RPT)])


def _compute_fidx(t1, s1, f1):
    @pl.loop(0, SCE // 16)
    def _(i):
        f1[pl.ds(i * 16, 16)] = (t1[pl.ds(i * 16, 16)] * N
                                 + s1[pl.ds(i * 16, 16)])


def _conv_body(scale, xrel_hbm, src3f_hbm, typ3f_hbm, dst3_hbm, sc3f_hbm,
               out_hbm, s1, t1, f1, d2, scb, rowsa, rowsb, agg, sema, semb):
    cc = lax.axis_index("c")
    s = lax.axis_index("s")
    wid = cc * NS + s

    _zero_rows(rowsa)
    _zero_agg(agg, rowsa, s)
    plsc.subcore_barrier()

    def start_g(c, buf, sem):
        pltpu.async_copy(xrel_hbm.at[f1.at[pl.ds(c * CH, CH)]], buf, sem)

    def wait_g(buf, sem):
        pltpu.make_async_copy(xrel_hbm.at[f1.at[pl.ds(0, CH)]], buf,
                              sem).wait()

    def consume(c, buf):
        if scale:
            @pl.loop(0, CH)
            def _(e):
                ssp = plsc.load_gather(scb, [jnp.full((16,), c * CH + e,
                                                      i32)])
                for j in range(D // 16):
                    buf[e, pl.ds(j * 16, 16)] = (
                        buf[e, pl.ds(j * 16, 16)] * ssp)
        pltpu.sync_copy(buf, agg.at[d2.at[c]], add=True)

    @pl.loop(0, NSC)
    def _(u):
        pltpu.sync_copy(src3f_hbm.at[wid, u], s1)
        pltpu.sync_copy(typ3f_hbm.at[wid, u], t1)
        pltpu.sync_copy(dst3_hbm.at[wid, u], d2)
        if scale:
            pltpu.sync_copy(sc3f_hbm.at[wid, u], scb)
        _compute_fidx(t1, s1, f1)

        start_g(0, rowsa, sema)

        @pl.loop(0, (NCH - 1) // 2)
        def _(t):
            start_g(2 * t + 1, rowsb, semb)
            wait_g(rowsa, sema)
            consume(2 * t, rowsa)
            start_g(2 * t + 2, rowsa, sema)
            wait_g(rowsb, semb)
            consume(2 * t + 1, rowsb)

        wait_g(rowsa, sema)
        consume(NCH - 1, rowsa)

    plsc.subcore_barrier()
    _writeout_agg(agg, out_hbm, cc, s)


def _conv_sc(xrel, src3f, typ3f, dst3, sc3f, scale):
    mesh = plsc.VectorSubcoreMesh(core_axis_name="c", subcore_axis_name="s")

    def body(xrel_, src_, typ_, dst_, sc_, out_, s1, t1, f1, d2, scb, rowsa,
             rowsb, agg, sema, semb):
        _conv_body(scale, xrel_, src_, typ_, dst_, sc_, out_, s1, t1, f1,
                   d2, scb, rowsa, rowsb, agg, sema, semb)

    return pl.kernel(
        body,
        out_type=jax.ShapeDtypeStruct((NC, N, D), f32),
        mesh=mesh,
        scratch_types=[
            pltpu.VMEM((SCE,), i32),
            pltpu.VMEM((SCE,), i32),
            pltpu.VMEM((SCE,), i32),
            pltpu.VMEM((NCH, CH), i32),
            pltpu.VMEM((SCE,), f32),
            pltpu.VMEM((CH, D), f32),
            pltpu.VMEM((CH, D), f32),
            pltpu.VMEM_SHARED((N, D), f32),
            pltpu.SemaphoreType.DMA,
            pltpu.SemaphoreType.DMA,
        ],
        **_SC_PARAMS,
    )(xrel, src3f, typ3f, dst3, sc3f)


def _k4g_body(x_hbm, src3f_hbm, dst3f_hbm, hs_hbm, ht_hbm,
              s1, d1, asr, acr, bsr, bcr, sa1, sa2, sb1, sb2):
    cc = lax.axis_index("c")
    s = lax.axis_index("s")
    wid = cc * NS + s

    def start_g(c, idx, buf, sem):
        pltpu.async_copy(x_hbm.at[idx.at[pl.ds(c * CH, CH)]], buf, sem)

    def wait_g(buf, sem):
        pltpu.make_async_copy(x_hbm.at[s1.at[pl.ds(0, CH)]], buf, sem).wait()

    @pl.loop(0, NSC)
    def _(u):
        pltpu.sync_copy(src3f_hbm.at[wid, u], s1)
        pltpu.sync_copy(dst3f_hbm.at[wid, u], d1)
        base = wid * EPW + u * SCE

        def consume(c, sbuf, cbuf):
            pltpu.sync_copy(sbuf, hs_hbm.at[pl.ds(base + c * CH, CH)])
            pltpu.sync_copy(cbuf, ht_hbm.at[pl.ds(base + c * CH, CH)])

        start_g(0, s1, asr, sa1)
        start_g(0, d1, acr, sa2)

        @pl.loop(0, (NCH - 1) // 2)
        def _(t):
            start_g(2 * t + 1, s1, bsr, sb1)
            start_g(2 * t + 1, d1, bcr, sb2)
            wait_g(asr, sa1)
            wait_g(acr, sa2)
            consume(2 * t, asr, acr)
            start_g(2 * t + 2, s1, asr, sa1)
            start_g(2 * t + 2, d1, acr, sa2)
            wait_g(bsr, sb1)
            wait_g(bcr, sb2)
            consume(2 * t + 1, bsr, bcr)

        wait_g(asr, sa1)
        wait_g(acr, sa2)
        consume(NCH - 1, asr, acr)


def _k4g(x, src3f, dst3f):
    mesh = plsc.VectorSubcoreMesh(core_axis_name="c", subcore_axis_name="s")
    return pl.kernel(
        _k4g_body,
        out_type=[
            jax.ShapeDtypeStruct((E, D), f32),
            jax.ShapeDtypeStruct((E, D), f32),
        ],
        mesh=mesh,
        scratch_types=[
            pltpu.VMEM((SCE,), i32),
            pltpu.VMEM((SCE,), i32),
            pltpu.VMEM((CH, D), f32),
            pltpu.VMEM((CH, D), f32),
            pltpu.VMEM((CH, D), f32),
            pltpu.VMEM((CH, D), f32),
            pltpu.SemaphoreType.DMA,
            pltpu.SemaphoreType.DMA,
            pltpu.SemaphoreType.DMA,
            pltpu.SemaphoreType.DMA,
        ],
        **_SC_PARAMS,
    )(x, src3f, dst3f)


EB = 3200
NEB = E // EB
EBR = EB // D


def _k4m_body(hs_ref, ht_ref, typ_ref, r_ref, gthr_ref, w0t_ref, b0_ref,
              w1t_ref, b1_ref, score_ref, gsm_ref):
    typ = jnp.min(typ_ref[...], axis=1, keepdims=True)
    r = r_ref[...]
    hr = jnp.zeros((EB, D), f32)
    for k in range(R2):
        hr = jnp.where(typ == k, r[k][None, :], hr)
    h = jnp.concatenate([hs_ref[...], hr, ht_ref[...]], axis=1)
    nrm = jnp.sqrt(jnp.sum(h * h, axis=1, keepdims=True))
    h = h / jnp.maximum(nrm, 1e-12)
    h2 = jax.nn.relu(jnp.dot(h, w0t_ref[...]) + b0_ref[...])
    z = jnp.dot(h2, w1t_ref[...]) + b1_ref[...]
    sc = jax.nn.sigmoid(z)
    gthr_c = jnp.min(gthr_ref[...], axis=1, keepdims=True)
    score_ref[...] = jnp.broadcast_to(sc, (EB, 8))
    gsm_ref[...] = jnp.broadcast_to(
        jnp.where(2.0 * sc - 1.0 >= gthr_c, 1.0, 0.0), (EB, 8))


def _k4m(hs, ht, typ8, r_enc, gthr8, w0t, b0, w1t, b1):
    full = lambda shp: pl.BlockSpec(shp, lambda n: tuple(0 for _ in shp))
    return pl.pallas_call(
        _k4m_body,
        grid=(NEB,),
        in_specs=[
            pl.BlockSpec((EB, D), lambda n: (n, 0)),
            pl.BlockSpec((EB, D), lambda n: (n, 0)),
            pl.BlockSpec((EB, 8), lambda n: (n, 0)),
            full((R2, D)),
            pl.BlockSpec((EB, 8), lambda n: (n, 0)),
            full((3 * D, D)),
            full((1, D)),
            full((D, 1)),
            full((1, 1)),
        ],
        out_specs=[
            pl.BlockSpec((EB, 8), lambda n: (n, 0)),
            pl.BlockSpec((EB, 8), lambda n: (n, 0)),
        ],
        out_shape=[
            jax.ShapeDtypeStruct((E, 8), f32),
            jax.ShapeDtypeStruct((E, 8), f32),
        ],
    )(hs, ht, typ8, r_enc, gthr8, w0t, b0, w1t, b1)


def kernel(edge_index, edge_type, init_embed, init_rel, w_rel, enc_Wrel,
           enc_Wroot, enc_b, lin0_w, lin0_b, lin1_w, lin1_b, dec_init_embed,
           dec_init_rel, dec_w_rel, dec_Wrel, dec_Wroot, dec_b):
    src3f = edge_index[0].reshape(NW, NSC, SCE)
    dst3f = edge_index[1].reshape(NW, NSC, SCE)
    typ3f = edge_type.reshape(NW, NSC, SCE)
    dst3 = edge_index[1].reshape(NW, NSC, NCH, CH)

    u = jax.random.uniform(jax.random.key(12345), (2, E, 1),
                           minval=1e-6, maxval=1.0 - 1e-6)
    u0 = u[0, :, 0].reshape(E // D, D)
    u1 = u[1, :, 0].reshape(E // D, D)

    xrel_enc, xrel_dec = _k1a(init_embed, enc_Wrel, dec_init_embed, dec_Wrel)
    gthr, r_, r_enc = _k1c(u0, u1, init_rel, w_rel, dec_init_rel, dec_w_rel)

    dummy_sc = jnp.zeros((8,), f32)
    parts_e = _conv_sc(xrel_enc, src3f, typ3f, dst3, dummy_sc, scale=False)
    x = _k3(parts_e[0], parts_e[1], init_embed, enc_Wroot,
            enc_b.reshape(1, D))

    hs, ht = _k4g(x, src3f, dst3f)
    typ8 = jnp.broadcast_to(edge_type.reshape(E, 1), (E, 8))
    gthr8 = jnp.broadcast_to(gthr.reshape(E, 1), (E, 8))
    score2, gsm2 = _k4m(hs, ht, typ8, r_enc, gthr8,
                        lin0_w.T, lin0_b.reshape(1, D), lin1_w.T,
                        lin1_b.reshape(1, 1))
    score3f = score2[:, 0].reshape(NW, NSC, SCE)

    parts_d = _conv_sc(xrel_dec, src3f, typ3f, dst3, score3f, scale=True)

    x_ = _k5(parts_d[0], parts_d[1], dec_init_embed, dec_Wroot,
             dec_b.reshape(1, D))

    return (x_, r_, gsm2[:, :1], score2[:, :1])
